# Initial kernel scaffold; baseline (speedup 1.0000x reference)
import jax
import jax.numpy as jnp
from jax.experimental import pallas as pl


def kernel(h, qproj_w, qproj_b, ln_g, ln_b, keys, values, res_w, res_b, mem_w, mem_b):
    raise NotImplementedError("write your pallas kernel here")


# speedup vs baseline: 17.3558x; 17.3558x over previous
import functools
import math

import jax
import jax.numpy as jnp
from jax.experimental import pallas as pl
from jax.experimental.pallas import tpu as pltpu

N_HEAD = 12
K_DIM = 16
V_DIM = 32
D_HEAD = 64
N_KEYS = 512
TOP_K = 8
NUM_SLOTS = 24
N_SLOT_TOT = NUM_SLOTS + N_HEAD
SCALE = 1.0 / math.sqrt(D_HEAD)
EPS = 1e-5

ROW_BLK = 256
CHUNK = 128


def _pkm_body(x_ref, w_ref, b_ref, g_ref, bln_ref, G_ref, Gt_ref,
              keys_ref, values_ref, rw_ref, rb_ref, mw_ref, mb_ref, out_ref):
    x = x_ref[...]
    q = jnp.dot(x, w_ref[...], preferred_element_type=jnp.float32) + b_ref[...]
    G = G_ref[...]
    s1 = jnp.dot(q, G, preferred_element_type=jnp.float32) * (1.0 / K_DIM)
    s2 = jnp.dot(q * q, G, preferred_element_type=jnp.float32) * (1.0 / K_DIM)
    var = s2 - s1 * s1
    Gt = Gt_ref[...]
    mu_b = jnp.dot(s1, Gt, preferred_element_type=jnp.float32)
    var_b = jnp.dot(var, Gt, preferred_element_type=jnp.float32)
    qn = (q - mu_b) * jax.lax.rsqrt(var_b + 1e-5) * g_ref[...] + bln_ref[...]

    for n in range(N_SLOT_TOT):
        ksrc = n if n < NUM_SLOTS else n - NUM_SLOTS
        qn_n = qn[:, K_DIM * n:K_DIM * (n + 1)]
        kk = keys_ref[ksrc]
        kk = kk * jax.lax.rsqrt(jnp.sum(kk * kk, axis=1, keepdims=True))
        sc = jax.lax.dot_general(qn_n, kk, (((1,), (1,)), ((), ())),
                                 preferred_element_type=jnp.float32)
        work = sc
        m0 = jnp.max(work, axis=1, keepdims=True)
        thr = m0
        for _ in range(TOP_K - 1):
            work = jnp.where(work >= thr, -1e30, work)
            thr = jnp.max(work, axis=1, keepdims=True)
        w = jnp.where(sc >= thr, jnp.exp(sc - m0), 0.0)
        z = jnp.sum(w, axis=1, keepdims=True)
        avg = jax.lax.dot_general(w, values_ref[ksrc], (((1,), (0,)), ((), ())),
                                  preferred_element_type=jnp.float32) / z
        res = jnp.dot(qn_n, rw_ref[...], preferred_element_type=jnp.float32) + rb_ref[...]
        o = jnp.dot(avg + res, mw_ref[...], preferred_element_type=jnp.float32) + mb_ref[...]
        out_ref[n] = o


def _attn_body(k_ref, v_ref, q_ref, o_ref, S_ref, z_ref):
    c = pl.program_id(1)

    @pl.when(c == 0)
    def _init():
        S_ref[...] = jnp.zeros_like(S_ref)
        z_ref[...] = jnp.zeros_like(z_ref)

    kk = k_ref[0]
    vv = v_ref[0]
    qq = q_ref[0]
    kk = jnp.where(kk > 0, kk + 1.0, jnp.exp(kk))
    qq = jnp.where(qq > 0, qq + 1.0, jnp.exp(qq))

    r = jax.lax.broadcasted_iota(jnp.int32, (CHUNK, CHUNK), 0)
    s = jax.lax.broadcasted_iota(jnp.int32, (CHUNK, CHUNK), 1)
    tril = (r >= s).astype(jnp.float32)

    csum_k = jnp.dot(tril, kk, preferred_element_type=jnp.float32)
    z0 = z_ref[0:1, :]
    denom = jnp.sum(qq * (z0 + csum_k), axis=1, keepdims=True)

    att = jax.lax.dot_general(qq, kk, (((1,), (1,)), ((), ())),
                              preferred_element_type=jnp.float32)
    att = att * tril
    out = (jnp.dot(att, vv, preferred_element_type=jnp.float32)
           + jnp.dot(qq, S_ref[...], preferred_element_type=jnp.float32))
    o_ref[0] = SCALE * out / (denom + EPS)

    S_ref[...] += jax.lax.dot_general(kk, vv, (((0,), (0,)), ((), ())),
                                      preferred_element_type=jnp.float32)
    z_ref[0:1, :] += jnp.sum(kk, axis=0, keepdims=True)


@jax.jit
def _run(x, qproj_w, qproj_b, g576, b576, G, Gt, keys, values,
         res_w, res_b, mem_w, mem_b):
    bs, d_model = x.shape
    n_blk = bs // ROW_BLK

    out36 = pl.pallas_call(
        _pkm_body,
        grid=(n_blk,),
        in_specs=[
            pl.BlockSpec((ROW_BLK, d_model), lambda i: (i, 0)),
            pl.BlockSpec((d_model, N_SLOT_TOT * K_DIM), lambda i: (0, 0)),
            pl.BlockSpec((1, N_SLOT_TOT * K_DIM), lambda i: (0, 0)),
            pl.BlockSpec((1, N_SLOT_TOT * K_DIM), lambda i: (0, 0)),
            pl.BlockSpec((1, N_SLOT_TOT * K_DIM), lambda i: (0, 0)),
            pl.BlockSpec((N_SLOT_TOT * K_DIM, N_SLOT_TOT), lambda i: (0, 0)),
            pl.BlockSpec((N_SLOT_TOT, N_SLOT_TOT * K_DIM), lambda i: (0, 0)),
            pl.BlockSpec((NUM_SLOTS, N_KEYS, K_DIM), lambda i: (0, 0, 0)),
            pl.BlockSpec((NUM_SLOTS, N_KEYS, V_DIM), lambda i: (0, 0, 0)),
            pl.BlockSpec((K_DIM, V_DIM), lambda i: (0, 0)),
            pl.BlockSpec((1, V_DIM), lambda i: (0, 0)),
            pl.BlockSpec((V_DIM, D_HEAD), lambda i: (0, 0)),
            pl.BlockSpec((1, D_HEAD), lambda i: (0, 0)),
        ],
        out_specs=pl.BlockSpec((N_SLOT_TOT, ROW_BLK, D_HEAD), lambda i: (0, i, 0)),
        out_shape=jax.ShapeDtypeStruct((N_SLOT_TOT, bs, D_HEAD), jnp.float32),
    )(x, qproj_w, qproj_b, g576, b576, G, Gt, keys, values,
      res_w, res_b, mem_w, mem_b)

    n_chunk = bs // CHUNK
    lo = pl.pallas_call(
        _attn_body,
        grid=(N_HEAD, n_chunk),
        in_specs=[
            pl.BlockSpec((1, CHUNK, D_HEAD), lambda h, c: (h, c, 0)),
            pl.BlockSpec((1, CHUNK, D_HEAD), lambda h, c: (h + N_HEAD, c, 0)),
            pl.BlockSpec((1, CHUNK, D_HEAD), lambda h, c: (h + 2 * N_HEAD, c, 0)),
        ],
        out_specs=pl.BlockSpec((1, CHUNK, D_HEAD), lambda h, c: (h, c, 0)),
        out_shape=jax.ShapeDtypeStruct((N_HEAD, bs, D_HEAD), jnp.float32),
        scratch_shapes=[
            pltpu.VMEM((D_HEAD, D_HEAD), jnp.float32),
            pltpu.VMEM((8, D_HEAD), jnp.float32),
        ],
        compiler_params=pltpu.CompilerParams(
            dimension_semantics=("arbitrary", "arbitrary"),
        ),
    )(out36, out36, out36)
    return lo


def kernel(h, qproj_w, qproj_b, ln_g, ln_b, keys, values, res_w, res_b,
           mem_w, mem_b):
    slen, bsz, d_model = h.shape
    x = h.reshape(slen * bsz, d_model)
    cols = N_SLOT_TOT * K_DIM
    grp = jnp.arange(cols, dtype=jnp.int32) // K_DIM
    G = (grp[:, None] == jnp.arange(N_SLOT_TOT, dtype=jnp.int32)[None, :]
         ).astype(jnp.float32)
    Gt = G.T
    g576 = jnp.tile(ln_g, N_SLOT_TOT)[None, :]
    b576 = jnp.tile(ln_b, N_SLOT_TOT)[None, :]
    lo = _run(x, qproj_w, qproj_b[None, :], g576, b576, G, Gt, keys, values,
              res_w, res_b[None, :], mem_w, mem_b[None, :])
    return lo.transpose(1, 0, 2).reshape(bsz, slen, N_HEAD, D_HEAD)


# speedup vs baseline: 24.0350x; 1.3848x over previous
import functools
import math

import jax
import jax.numpy as jnp
from jax.experimental import pallas as pl
from jax.experimental.pallas import tpu as pltpu

N_HEAD = 12
K_DIM = 16
V_DIM = 32
D_HEAD = 64
N_KEYS = 512
TOP_K = 8
NUM_SLOTS = 24
N_SLOT_TOT = NUM_SLOTS + N_HEAD
SCALE = 1.0 / math.sqrt(D_HEAD)
EPS = 1e-5

ROW_BLK = 256
CHUNK = 256


def _elu1(x):
    return jnp.where(x > 0, x + 1.0, jnp.exp(x))


def _pkm_body(x_ref, w_ref, b_ref, g_ref, bln_ref, G_ref, Gt_ref,
              keys_ref, values_ref, rw_ref, rb_ref, mw_ref, mb_ref, out_ref,
              avg_ref, q3_ref):
    x = x_ref[...]
    q = jnp.dot(x, w_ref[...], preferred_element_type=jnp.float32) + b_ref[...]
    G = G_ref[...]
    s1 = jnp.dot(q, G, preferred_element_type=jnp.float32) * (1.0 / K_DIM)
    s2 = jnp.dot(q * q, G, preferred_element_type=jnp.float32) * (1.0 / K_DIM)
    var = s2 - s1 * s1
    Gt = Gt_ref[...]
    mu_b = jnp.dot(s1, Gt, preferred_element_type=jnp.float32)
    var_b = jnp.dot(var, Gt, preferred_element_type=jnp.float32)
    qn = (q - mu_b) * jax.lax.rsqrt(var_b + 1e-5) * g_ref[...] + bln_ref[...]

    for n in range(N_SLOT_TOT):
        ksrc = n if n < NUM_SLOTS else n - NUM_SLOTS
        qn_n = qn[:, K_DIM * n:K_DIM * (n + 1)]
        kk = keys_ref[ksrc]
        kk = kk * jax.lax.rsqrt(jnp.sum(kk * kk, axis=1, keepdims=True))
        sc = jax.lax.dot_general(qn_n, kk, (((1,), (1,)), ((), ())),
                                 preferred_element_type=jnp.float32)
        work = sc
        m0 = jnp.max(work, axis=1, keepdims=True)
        thr = m0
        for _ in range(TOP_K - 1):
            work = jnp.where(work >= thr, -1e30, work)
            thr = jnp.max(work, axis=1, keepdims=True)
        w = jnp.where(sc >= thr, jnp.exp(sc - m0), 0.0)
        z = jnp.sum(w, axis=1, keepdims=True)
        avg_ref[n] = jax.lax.dot_general(
            w, values_ref[ksrc], (((1,), (0,)), ((), ())),
            preferred_element_type=jnp.float32) / z
        q3_ref[n] = qn_n

    avg = avg_ref[...].reshape(N_SLOT_TOT * ROW_BLK, V_DIM)
    q3 = q3_ref[...].reshape(N_SLOT_TOT * ROW_BLK, K_DIM)
    res = jnp.dot(q3, rw_ref[...], preferred_element_type=jnp.float32) + rb_ref[...]
    o = jnp.dot(avg + res, mw_ref[...], preferred_element_type=jnp.float32) + mb_ref[...]
    o = o.reshape(N_SLOT_TOT, ROW_BLK, D_HEAD)
    out_ref[0:N_HEAD] = _elu1(o[0:N_HEAD])
    out_ref[N_HEAD:NUM_SLOTS] = o[N_HEAD:NUM_SLOTS]
    out_ref[NUM_SLOTS:] = _elu1(o[NUM_SLOTS:])


def _attn_body(k_ref, v_ref, q_ref, o_ref, S_ref, z_ref):
    c = pl.program_id(0)

    @pl.when(c == 0)
    def _init():
        S_ref[...] = jnp.zeros_like(S_ref)
        z_ref[...] = jnp.zeros_like(z_ref)

    r = jax.lax.broadcasted_iota(jnp.int32, (CHUNK, CHUNK), 0)
    s = jax.lax.broadcasted_iota(jnp.int32, (CHUNK, CHUNK), 1)
    tril = (r >= s).astype(jnp.float32)

    for h in range(N_HEAD):
        kk = k_ref[h]
        vv = v_ref[h]
        qq = q_ref[h]
        att = jax.lax.dot_general(qq, kk, (((1,), (1,)), ((), ())),
                                  preferred_element_type=jnp.float32)
        att = att * tril
        z0 = z_ref[h, 0:1, :]
        denom = (jnp.sum(att, axis=1, keepdims=True)
                 + jnp.sum(qq * z0, axis=1, keepdims=True))
        out = (jnp.dot(att, vv, preferred_element_type=jnp.float32)
               + jnp.dot(qq, S_ref[h], preferred_element_type=jnp.float32))
        o_ref[:, h, :] = SCALE * out / (denom + EPS)
        S_ref[h] += jax.lax.dot_general(kk, vv, (((0,), (0,)), ((), ())),
                                        preferred_element_type=jnp.float32)
        z_ref[h, 0:1, :] += jnp.sum(kk, axis=0, keepdims=True)


@jax.jit
def _run(x, qproj_w, qproj_b, g576, b576, G, Gt, keys, values,
         res_w, res_b, mem_w, mem_b):
    bs, d_model = x.shape
    n_blk = bs // ROW_BLK

    out36 = pl.pallas_call(
        _pkm_body,
        grid=(n_blk,),
        in_specs=[
            pl.BlockSpec((ROW_BLK, d_model), lambda i: (i, 0)),
            pl.BlockSpec((d_model, N_SLOT_TOT * K_DIM), lambda i: (0, 0)),
            pl.BlockSpec((1, N_SLOT_TOT * K_DIM), lambda i: (0, 0)),
            pl.BlockSpec((1, N_SLOT_TOT * K_DIM), lambda i: (0, 0)),
            pl.BlockSpec((1, N_SLOT_TOT * K_DIM), lambda i: (0, 0)),
            pl.BlockSpec((N_SLOT_TOT * K_DIM, N_SLOT_TOT), lambda i: (0, 0)),
            pl.BlockSpec((N_SLOT_TOT, N_SLOT_TOT * K_DIM), lambda i: (0, 0)),
            pl.BlockSpec((NUM_SLOTS, N_KEYS, K_DIM), lambda i: (0, 0, 0)),
            pl.BlockSpec((NUM_SLOTS, N_KEYS, V_DIM), lambda i: (0, 0, 0)),
            pl.BlockSpec((K_DIM, V_DIM), lambda i: (0, 0)),
            pl.BlockSpec((1, V_DIM), lambda i: (0, 0)),
            pl.BlockSpec((V_DIM, D_HEAD), lambda i: (0, 0)),
            pl.BlockSpec((1, D_HEAD), lambda i: (0, 0)),
        ],
        out_specs=pl.BlockSpec((N_SLOT_TOT, ROW_BLK, D_HEAD), lambda i: (0, i, 0)),
        out_shape=jax.ShapeDtypeStruct((N_SLOT_TOT, bs, D_HEAD), jnp.float32),
        scratch_shapes=[
            pltpu.VMEM((N_SLOT_TOT, ROW_BLK, V_DIM), jnp.float32),
            pltpu.VMEM((N_SLOT_TOT, ROW_BLK, K_DIM), jnp.float32),
        ],
    )(x, qproj_w, qproj_b, g576, b576, G, Gt, keys, values,
      res_w, res_b, mem_w, mem_b)

    n_chunk = bs // CHUNK
    lo = pl.pallas_call(
        _attn_body,
        grid=(n_chunk,),
        in_specs=[
            pl.BlockSpec((N_HEAD, CHUNK, D_HEAD), lambda c: (0, c, 0)),
            pl.BlockSpec((N_HEAD, CHUNK, D_HEAD), lambda c: (1, c, 0)),
            pl.BlockSpec((N_HEAD, CHUNK, D_HEAD), lambda c: (2, c, 0)),
        ],
        out_specs=pl.BlockSpec((CHUNK, N_HEAD, D_HEAD), lambda c: (c, 0, 0)),
        out_shape=jax.ShapeDtypeStruct((bs, N_HEAD, D_HEAD), jnp.float32),
        scratch_shapes=[
            pltpu.VMEM((N_HEAD, D_HEAD, D_HEAD), jnp.float32),
            pltpu.VMEM((N_HEAD, 8, D_HEAD), jnp.float32),
        ],
        compiler_params=pltpu.CompilerParams(
            dimension_semantics=("arbitrary",),
        ),
    )(out36, out36, out36)
    return lo


def kernel(h, qproj_w, qproj_b, ln_g, ln_b, keys, values, res_w, res_b,
           mem_w, mem_b):
    slen, bsz, d_model = h.shape
    x = h.reshape(slen * bsz, d_model)
    cols = N_SLOT_TOT * K_DIM
    grp = jnp.arange(cols, dtype=jnp.int32) // K_DIM
    G = (grp[:, None] == jnp.arange(N_SLOT_TOT, dtype=jnp.int32)[None, :]
         ).astype(jnp.float32)
    Gt = G.T
    g576 = jnp.tile(ln_g, N_SLOT_TOT)[None, :]
    b576 = jnp.tile(ln_b, N_SLOT_TOT)[None, :]
    lo = _run(x, qproj_w, qproj_b[None, :], g576, b576, G, Gt, keys, values,
              res_w, res_b[None, :], mem_w, mem_b[None, :])
    return lo.reshape(bsz, slen, N_HEAD, D_HEAD)


# speedup vs baseline: 24.0403x; 1.0002x over previous
import functools
import math

import jax
import jax.numpy as jnp
from jax.experimental import pallas as pl
from jax.experimental.pallas import tpu as pltpu

N_HEAD = 12
K_DIM = 16
V_DIM = 32
D_HEAD = 64
N_KEYS = 512
TOP_K = 8
NUM_SLOTS = 24
N_SLOT_TOT = NUM_SLOTS + N_HEAD
SCALE = 1.0 / math.sqrt(D_HEAD)
EPS = 1e-5

ROW_BLK = 256
CHUNK = 256


def _elu1(x):
    return jnp.where(x > 0, x + 1.0, jnp.exp(x))


def _pkm_body(x_ref, w_ref, b_ref, g_ref, bln_ref, G_ref, Gt_ref,
              keys_ref, values_ref, rw_ref, rb_ref, mw_ref, mb_ref, out_ref,
              avg_ref, q3_ref):
    x = x_ref[...]
    q = jnp.dot(x, w_ref[...], preferred_element_type=jnp.float32) + b_ref[...]
    G = G_ref[...]
    s1 = jnp.dot(q, G, preferred_element_type=jnp.float32) * (1.0 / K_DIM)
    s2 = jnp.dot(q * q, G, preferred_element_type=jnp.float32) * (1.0 / K_DIM)
    var = s2 - s1 * s1
    Gt = Gt_ref[...]
    mu_b = jnp.dot(s1, Gt, preferred_element_type=jnp.float32)
    var_b = jnp.dot(var, Gt, preferred_element_type=jnp.float32)
    qn = (q - mu_b) * jax.lax.rsqrt(var_b + 1e-5) * g_ref[...] + bln_ref[...]

    Q = N_KEYS // 4
    for n in range(N_SLOT_TOT):
        ksrc = n if n < NUM_SLOTS else n - NUM_SLOTS
        qn_n = qn[:, K_DIM * n:K_DIM * (n + 1)]
        kk = keys_ref[ksrc]
        kk = kk * jax.lax.rsqrt(jnp.sum(kk * kk, axis=1, keepdims=True))
        sc = jax.lax.dot_general(qn_n, kk, (((1,), (1,)), ((), ())),
                                 preferred_element_type=jnp.float32)
        a, b = sc[:, 0:Q], sc[:, Q:2 * Q]
        cc, d = sc[:, 2 * Q:3 * Q], sc[:, 3 * Q:]
        ab_h, ab_l = jnp.maximum(a, b), jnp.minimum(a, b)
        cd_h, cd_l = jnp.maximum(cc, d), jnp.minimum(cc, d)
        t1 = jnp.maximum(ab_h, cd_h)
        m_h = jnp.minimum(ab_h, cd_h)
        t4 = jnp.minimum(ab_l, cd_l)
        m_l = jnp.maximum(ab_l, cd_l)
        t2 = jnp.maximum(m_h, m_l)
        t3 = jnp.minimum(m_h, m_l)
        m0 = jnp.max(t1, axis=1, keepdims=True)
        thr = m0
        for _ in range(TOP_K - 1):
            e = t1 >= thr
            t1 = jnp.where(e, t2, t1)
            t2 = jnp.where(e, t3, t2)
            t3 = jnp.where(e, t4, t3)
            t4 = jnp.where(e, -1e30, t4)
            thr = jnp.max(t1, axis=1, keepdims=True)
        w = jnp.where(sc >= thr, jnp.exp(sc - m0), 0.0)
        z = jnp.sum(w, axis=1, keepdims=True)
        avg_ref[n] = jax.lax.dot_general(
            w, values_ref[ksrc], (((1,), (0,)), ((), ())),
            preferred_element_type=jnp.float32) / z
        q3_ref[n] = qn_n

    avg = avg_ref[...].reshape(N_SLOT_TOT * ROW_BLK, V_DIM)
    q3 = q3_ref[...].reshape(N_SLOT_TOT * ROW_BLK, K_DIM)
    res = jnp.dot(q3, rw_ref[...], preferred_element_type=jnp.float32) + rb_ref[...]
    o = jnp.dot(avg + res, mw_ref[...], preferred_element_type=jnp.float32) + mb_ref[...]
    o = o.reshape(N_SLOT_TOT, ROW_BLK, D_HEAD)
    out_ref[0:N_HEAD] = _elu1(o[0:N_HEAD])
    out_ref[N_HEAD:NUM_SLOTS] = o[N_HEAD:NUM_SLOTS]
    out_ref[NUM_SLOTS:] = _elu1(o[NUM_SLOTS:])


def _attn_body(k_ref, v_ref, q_ref, o_ref, S_ref, z_ref):
    c = pl.program_id(0)

    @pl.when(c == 0)
    def _init():
        S_ref[...] = jnp.zeros_like(S_ref)
        z_ref[...] = jnp.zeros_like(z_ref)

    r = jax.lax.broadcasted_iota(jnp.int32, (CHUNK, CHUNK), 0)
    s = jax.lax.broadcasted_iota(jnp.int32, (CHUNK, CHUNK), 1)
    tril = (r >= s).astype(jnp.float32)

    for h in range(N_HEAD):
        kk = k_ref[h]
        vv = v_ref[h]
        qq = q_ref[h]
        att = jax.lax.dot_general(qq, kk, (((1,), (1,)), ((), ())),
                                  preferred_element_type=jnp.float32)
        att = att * tril
        z0 = z_ref[h, 0:1, :]
        denom = (jnp.sum(att, axis=1, keepdims=True)
                 + jnp.sum(qq * z0, axis=1, keepdims=True))
        out = (jnp.dot(att, vv, preferred_element_type=jnp.float32)
               + jnp.dot(qq, S_ref[h], preferred_element_type=jnp.float32))
        o_ref[:, h, :] = SCALE * out / (denom + EPS)
        S_ref[h] += jax.lax.dot_general(kk, vv, (((0,), (0,)), ((), ())),
                                        preferred_element_type=jnp.float32)
        z_ref[h, 0:1, :] += jnp.sum(kk, axis=0, keepdims=True)


@jax.jit
def _run(x, qproj_w, qproj_b, g576, b576, G, Gt, keys, values,
         res_w, res_b, mem_w, mem_b):
    bs, d_model = x.shape
    n_blk = bs // ROW_BLK

    out36 = pl.pallas_call(
        _pkm_body,
        grid=(n_blk,),
        in_specs=[
            pl.BlockSpec((ROW_BLK, d_model), lambda i: (i, 0)),
            pl.BlockSpec((d_model, N_SLOT_TOT * K_DIM), lambda i: (0, 0)),
            pl.BlockSpec((1, N_SLOT_TOT * K_DIM), lambda i: (0, 0)),
            pl.BlockSpec((1, N_SLOT_TOT * K_DIM), lambda i: (0, 0)),
            pl.BlockSpec((1, N_SLOT_TOT * K_DIM), lambda i: (0, 0)),
            pl.BlockSpec((N_SLOT_TOT * K_DIM, N_SLOT_TOT), lambda i: (0, 0)),
            pl.BlockSpec((N_SLOT_TOT, N_SLOT_TOT * K_DIM), lambda i: (0, 0)),
            pl.BlockSpec((NUM_SLOTS, N_KEYS, K_DIM), lambda i: (0, 0, 0)),
            pl.BlockSpec((NUM_SLOTS, N_KEYS, V_DIM), lambda i: (0, 0, 0)),
            pl.BlockSpec((K_DIM, V_DIM), lambda i: (0, 0)),
            pl.BlockSpec((1, V_DIM), lambda i: (0, 0)),
            pl.BlockSpec((V_DIM, D_HEAD), lambda i: (0, 0)),
            pl.BlockSpec((1, D_HEAD), lambda i: (0, 0)),
        ],
        out_specs=pl.BlockSpec((N_SLOT_TOT, ROW_BLK, D_HEAD), lambda i: (0, i, 0)),
        out_shape=jax.ShapeDtypeStruct((N_SLOT_TOT, bs, D_HEAD), jnp.float32),
        scratch_shapes=[
            pltpu.VMEM((N_SLOT_TOT, ROW_BLK, V_DIM), jnp.float32),
            pltpu.VMEM((N_SLOT_TOT, ROW_BLK, K_DIM), jnp.float32),
        ],
    )(x, qproj_w, qproj_b, g576, b576, G, Gt, keys, values,
      res_w, res_b, mem_w, mem_b)

    n_chunk = bs // CHUNK
    lo = pl.pallas_call(
        _attn_body,
        grid=(n_chunk,),
        in_specs=[
            pl.BlockSpec((N_HEAD, CHUNK, D_HEAD), lambda c: (0, c, 0)),
            pl.BlockSpec((N_HEAD, CHUNK, D_HEAD), lambda c: (1, c, 0)),
            pl.BlockSpec((N_HEAD, CHUNK, D_HEAD), lambda c: (2, c, 0)),
        ],
        out_specs=pl.BlockSpec((CHUNK, N_HEAD, D_HEAD), lambda c: (c, 0, 0)),
        out_shape=jax.ShapeDtypeStruct((bs, N_HEAD, D_HEAD), jnp.float32),
        scratch_shapes=[
            pltpu.VMEM((N_HEAD, D_HEAD, D_HEAD), jnp.float32),
            pltpu.VMEM((N_HEAD, 8, D_HEAD), jnp.float32),
        ],
        compiler_params=pltpu.CompilerParams(
            dimension_semantics=("arbitrary",),
        ),
    )(out36, out36, out36)
    return lo


def kernel(h, qproj_w, qproj_b, ln_g, ln_b, keys, values, res_w, res_b,
           mem_w, mem_b):
    slen, bsz, d_model = h.shape
    x = h.reshape(slen * bsz, d_model)
    cols = N_SLOT_TOT * K_DIM
    grp = jnp.arange(cols, dtype=jnp.int32) // K_DIM
    G = (grp[:, None] == jnp.arange(N_SLOT_TOT, dtype=jnp.int32)[None, :]
         ).astype(jnp.float32)
    Gt = G.T
    g576 = jnp.tile(ln_g, N_SLOT_TOT)[None, :]
    b576 = jnp.tile(ln_b, N_SLOT_TOT)[None, :]
    lo = _run(x, qproj_w, qproj_b[None, :], g576, b576, G, Gt, keys, values,
              res_w, res_b[None, :], mem_w, mem_b[None, :])
    return lo.reshape(bsz, slen, N_HEAD, D_HEAD)


# speedup vs baseline: 27.5659x; 1.1467x over previous
import functools
import math

import jax
import jax.numpy as jnp
from jax.experimental import pallas as pl
from jax.experimental.pallas import tpu as pltpu

N_HEAD = 12
K_DIM = 16
V_DIM = 32
D_HEAD = 64
N_KEYS = 512
TOP_K = 8
NUM_SLOTS = 24
N_SLOT_TOT = NUM_SLOTS + N_HEAD
SCALE = 1.0 / math.sqrt(D_HEAD)
EPS = 1e-5

ROW_BLK = 256
CHUNK = 256


def _elu1(x):
    return jnp.where(x > 0, x + 1.0, jnp.exp(x))


def _pkm_body(x_ref, w_ref, b_ref, g_ref, bln_ref, G_ref, Gt_ref,
              keys_ref, values_ref, rw_ref, rb_ref, mw_ref, mb_ref, out_ref,
              avg_ref, q3_ref, kn_ref, scA_ref, scB_ref):
    x = x_ref[...]
    q = jnp.dot(x, w_ref[...], preferred_element_type=jnp.float32) + b_ref[...]
    G = G_ref[...]
    s1 = jnp.dot(q, G, preferred_element_type=jnp.float32) * (1.0 / K_DIM)
    s2 = jnp.dot(q * q, G, preferred_element_type=jnp.float32) * (1.0 / K_DIM)
    var = s2 - s1 * s1
    Gt = Gt_ref[...]
    mu_b = jnp.dot(s1, Gt, preferred_element_type=jnp.float32)
    var_b = jnp.dot(var, Gt, preferred_element_type=jnp.float32)
    qn = (q - mu_b) * jax.lax.rsqrt(var_b + 1e-5) * g_ref[...] + bln_ref[...]

    for k in range(NUM_SLOTS):
        kk = keys_ref[k]
        kn_ref[k] = kk * jax.lax.rsqrt(jnp.sum(kk * kk, axis=1, keepdims=True))

    Q = N_KEYS // 4
    sc_bufs = (scA_ref, scB_ref)

    def _scores(i):
        ksrc = i if i < NUM_SLOTS else i - NUM_SLOTS
        qn_i = qn[:, K_DIM * i:K_DIM * (i + 1)]
        sc_bufs[i % 2][...] = jax.lax.dot_general(
            qn_i, kn_ref[ksrc], (((1,), (1,)), ((), ())),
            preferred_element_type=jnp.float32)

    _scores(0)
    for n in range(N_SLOT_TOT):
        if n + 1 < N_SLOT_TOT:
            _scores(n + 1)
        sc = sc_bufs[n % 2][...]
        a, b = sc[:, 0:Q], sc[:, Q:2 * Q]
        cc, d = sc[:, 2 * Q:3 * Q], sc[:, 3 * Q:]
        ab_h, ab_l = jnp.maximum(a, b), jnp.minimum(a, b)
        cd_h, cd_l = jnp.maximum(cc, d), jnp.minimum(cc, d)
        t1 = jnp.maximum(ab_h, cd_h)
        m_h = jnp.minimum(ab_h, cd_h)
        t4 = jnp.minimum(ab_l, cd_l)
        m_l = jnp.maximum(ab_l, cd_l)
        t2 = jnp.maximum(m_h, m_l)
        t3 = jnp.minimum(m_h, m_l)
        m0 = jnp.max(t1, axis=1, keepdims=True)
        thr = m0
        for it in range(TOP_K - 1):
            e = t1 >= thr
            t1 = jnp.where(e, t2, t1)
            if it < TOP_K - 2:
                t2 = jnp.where(e, t3, t2)
            if it < TOP_K - 3:
                t3 = jnp.where(e, t4, t3)
            if it < TOP_K - 4:
                t4 = jnp.where(e, -1e30, t4)
            thr = jnp.max(t1, axis=1, keepdims=True)
        w = jnp.where(sc >= thr, jnp.exp(sc - m0), 0.0)
        z = jnp.sum(w, axis=1, keepdims=True)
        ksrc = n if n < NUM_SLOTS else n - NUM_SLOTS
        avg_ref[n] = jax.lax.dot_general(
            w, values_ref[ksrc], (((1,), (0,)), ((), ())),
            preferred_element_type=jnp.float32) / z
        q3_ref[n] = qn[:, K_DIM * n:K_DIM * (n + 1)]

    avg = avg_ref[...].reshape(N_SLOT_TOT * ROW_BLK, V_DIM)
    q3 = q3_ref[...].reshape(N_SLOT_TOT * ROW_BLK, K_DIM)
    res = jnp.dot(q3, rw_ref[...], preferred_element_type=jnp.float32) + rb_ref[...]
    o = jnp.dot(avg + res, mw_ref[...], preferred_element_type=jnp.float32) + mb_ref[...]
    o = o.reshape(N_SLOT_TOT, ROW_BLK, D_HEAD)
    out_ref[0:N_HEAD] = _elu1(o[0:N_HEAD])
    out_ref[N_HEAD:NUM_SLOTS] = o[N_HEAD:NUM_SLOTS]
    out_ref[NUM_SLOTS:] = _elu1(o[NUM_SLOTS:])


def _attn_body(k_ref, v_ref, q_ref, o_ref, S_ref, z_ref):
    c = pl.program_id(0)

    @pl.when(c == 0)
    def _init():
        S_ref[...] = jnp.zeros_like(S_ref)
        z_ref[...] = jnp.zeros_like(z_ref)

    r = jax.lax.broadcasted_iota(jnp.int32, (CHUNK, CHUNK), 0)
    s = jax.lax.broadcasted_iota(jnp.int32, (CHUNK, CHUNK), 1)
    tril = (r >= s).astype(jnp.float32)

    for h in range(N_HEAD):
        kk = k_ref[h]
        vv = v_ref[h]
        qq = q_ref[h]
        att = jax.lax.dot_general(qq, kk, (((1,), (1,)), ((), ())),
                                  preferred_element_type=jnp.float32)
        att = att * tril
        z0 = z_ref[h, 0:1, :]
        denom = (jnp.sum(att, axis=1, keepdims=True)
                 + jnp.sum(qq * z0, axis=1, keepdims=True))
        out = (jnp.dot(att, vv, preferred_element_type=jnp.float32)
               + jnp.dot(qq, S_ref[h], preferred_element_type=jnp.float32))
        o_ref[:, h, :] = SCALE * out / (denom + EPS)
        S_ref[h] += jax.lax.dot_general(kk, vv, (((0,), (0,)), ((), ())),
                                        preferred_element_type=jnp.float32)
        z_ref[h, 0:1, :] += jnp.sum(kk, axis=0, keepdims=True)


@jax.jit
def _run(x, qproj_w, qproj_b, g576, b576, G, Gt, keys, values,
         res_w, res_b, mem_w, mem_b):
    bs, d_model = x.shape
    n_blk = bs // ROW_BLK

    out36 = pl.pallas_call(
        _pkm_body,
        grid=(n_blk,),
        in_specs=[
            pl.BlockSpec((ROW_BLK, d_model), lambda i: (i, 0)),
            pl.BlockSpec((d_model, N_SLOT_TOT * K_DIM), lambda i: (0, 0)),
            pl.BlockSpec((1, N_SLOT_TOT * K_DIM), lambda i: (0, 0)),
            pl.BlockSpec((1, N_SLOT_TOT * K_DIM), lambda i: (0, 0)),
            pl.BlockSpec((1, N_SLOT_TOT * K_DIM), lambda i: (0, 0)),
            pl.BlockSpec((N_SLOT_TOT * K_DIM, N_SLOT_TOT), lambda i: (0, 0)),
            pl.BlockSpec((N_SLOT_TOT, N_SLOT_TOT * K_DIM), lambda i: (0, 0)),
            pl.BlockSpec((NUM_SLOTS, N_KEYS, K_DIM), lambda i: (0, 0, 0)),
            pl.BlockSpec((NUM_SLOTS, N_KEYS, V_DIM), lambda i: (0, 0, 0)),
            pl.BlockSpec((K_DIM, V_DIM), lambda i: (0, 0)),
            pl.BlockSpec((1, V_DIM), lambda i: (0, 0)),
            pl.BlockSpec((V_DIM, D_HEAD), lambda i: (0, 0)),
            pl.BlockSpec((1, D_HEAD), lambda i: (0, 0)),
        ],
        out_specs=pl.BlockSpec((N_SLOT_TOT, ROW_BLK, D_HEAD), lambda i: (0, i, 0)),
        out_shape=jax.ShapeDtypeStruct((N_SLOT_TOT, bs, D_HEAD), jnp.float32),
        scratch_shapes=[
            pltpu.VMEM((N_SLOT_TOT, ROW_BLK, V_DIM), jnp.float32),
            pltpu.VMEM((N_SLOT_TOT, ROW_BLK, K_DIM), jnp.float32),
            pltpu.VMEM((NUM_SLOTS, N_KEYS, K_DIM), jnp.float32),
            pltpu.VMEM((ROW_BLK, N_KEYS), jnp.float32),
            pltpu.VMEM((ROW_BLK, N_KEYS), jnp.float32),
        ],
    )(x, qproj_w, qproj_b, g576, b576, G, Gt, keys, values,
      res_w, res_b, mem_w, mem_b)

    n_chunk = bs // CHUNK
    lo = pl.pallas_call(
        _attn_body,
        grid=(n_chunk,),
        in_specs=[
            pl.BlockSpec((N_HEAD, CHUNK, D_HEAD), lambda c: (0, c, 0)),
            pl.BlockSpec((N_HEAD, CHUNK, D_HEAD), lambda c: (1, c, 0)),
            pl.BlockSpec((N_HEAD, CHUNK, D_HEAD), lambda c: (2, c, 0)),
        ],
        out_specs=pl.BlockSpec((CHUNK, N_HEAD, D_HEAD), lambda c: (c, 0, 0)),
        out_shape=jax.ShapeDtypeStruct((bs, N_HEAD, D_HEAD), jnp.float32),
        scratch_shapes=[
            pltpu.VMEM((N_HEAD, D_HEAD, D_HEAD), jnp.float32),
            pltpu.VMEM((N_HEAD, 8, D_HEAD), jnp.float32),
        ],
        compiler_params=pltpu.CompilerParams(
            dimension_semantics=("arbitrary",),
        ),
    )(out36, out36, out36)
    return lo


def kernel(h, qproj_w, qproj_b, ln_g, ln_b, keys, values, res_w, res_b,
           mem_w, mem_b):
    slen, bsz, d_model = h.shape
    x = h.reshape(slen * bsz, d_model)
    cols = N_SLOT_TOT * K_DIM
    grp = jnp.arange(cols, dtype=jnp.int32) // K_DIM
    G = (grp[:, None] == jnp.arange(N_SLOT_TOT, dtype=jnp.int32)[None, :]
         ).astype(jnp.float32)
    Gt = G.T
    g576 = jnp.tile(ln_g, N_SLOT_TOT)[None, :]
    b576 = jnp.tile(ln_b, N_SLOT_TOT)[None, :]
    lo = _run(x, qproj_w, qproj_b[None, :], g576, b576, G, Gt, keys, values,
              res_w, res_b[None, :], mem_w, mem_b[None, :])
    return lo.reshape(bsz, slen, N_HEAD, D_HEAD)


# speedup vs baseline: 28.5344x; 1.0351x over previous
import functools
import math

import jax
import jax.numpy as jnp
from jax.experimental import pallas as pl
from jax.experimental.pallas import tpu as pltpu

N_HEAD = 12
K_DIM = 16
V_DIM = 32
D_HEAD = 64
N_KEYS = 512
TOP_K = 8
NUM_SLOTS = 24
N_SLOT_TOT = NUM_SLOTS + N_HEAD
SCALE = 1.0 / math.sqrt(D_HEAD)
EPS = 1e-5

ROW_BLK = 256
CHUNK = 256


def _elu1(x):
    return jnp.where(x > 0, x + 1.0, jnp.exp(x))


def _pkm_body(x_ref, w_ref, b_ref, g_ref, bln_ref, G_ref, Gt_ref,
              keys_ref, values_ref, rw_ref, rb_ref, mw_ref, mb_ref, out_ref,
              avg_ref, q3_ref, kn_ref, scA_ref, scB_ref, S_ref, z_ref):
    x = x_ref[...]
    q = jnp.dot(x, w_ref[...], preferred_element_type=jnp.float32) + b_ref[...]
    G = G_ref[...]
    s1 = jnp.dot(q, G, preferred_element_type=jnp.float32) * (1.0 / K_DIM)
    s2 = jnp.dot(q * q, G, preferred_element_type=jnp.float32) * (1.0 / K_DIM)
    var = s2 - s1 * s1
    Gt = Gt_ref[...]
    mu_b = jnp.dot(s1, Gt, preferred_element_type=jnp.float32)
    var_b = jnp.dot(var, Gt, preferred_element_type=jnp.float32)
    qn = (q - mu_b) * jax.lax.rsqrt(var_b + 1e-5) * g_ref[...] + bln_ref[...]

    for k in range(NUM_SLOTS):
        kk = keys_ref[k]
        kn_ref[k] = kk * jax.lax.rsqrt(jnp.sum(kk * kk, axis=1, keepdims=True))

    Q = N_KEYS // 4
    sc_bufs = (scA_ref, scB_ref)

    def _scores(i):
        ksrc = i if i < NUM_SLOTS else i - NUM_SLOTS
        qn_i = qn[:, K_DIM * i:K_DIM * (i + 1)]
        sc_bufs[i % 2][...] = jax.lax.dot_general(
            qn_i, kn_ref[ksrc], (((1,), (1,)), ((), ())),
            preferred_element_type=jnp.float32)

    _scores(0)
    for n in range(N_SLOT_TOT):
        if n + 1 < N_SLOT_TOT:
            _scores(n + 1)
        sc = sc_bufs[n % 2][...]
        a, b = sc[:, 0:Q], sc[:, Q:2 * Q]
        cc, d = sc[:, 2 * Q:3 * Q], sc[:, 3 * Q:]
        ab_h, ab_l = jnp.maximum(a, b), jnp.minimum(a, b)
        cd_h, cd_l = jnp.maximum(cc, d), jnp.minimum(cc, d)
        t1 = jnp.maximum(ab_h, cd_h)
        m_h = jnp.minimum(ab_h, cd_h)
        t4 = jnp.minimum(ab_l, cd_l)
        m_l = jnp.maximum(ab_l, cd_l)
        t2 = jnp.maximum(m_h, m_l)
        t3 = jnp.minimum(m_h, m_l)
        m0 = jnp.max(t1, axis=1, keepdims=True)
        thr = m0
        for it in range(TOP_K - 1):
            e = t1 >= thr
            t1 = jnp.where(e, t2, t1)
            if it < TOP_K - 2:
                t2 = jnp.where(e, t3, t2)
            if it < TOP_K - 3:
                t3 = jnp.where(e, t4, t3)
            if it < TOP_K - 4:
                t4 = jnp.where(e, -1e30, t4)
            thr = jnp.max(t1, axis=1, keepdims=True)
        w = jnp.where(sc >= thr, jnp.exp(sc - m0), 0.0)
        z = jnp.sum(w, axis=1, keepdims=True)
        ksrc = n if n < NUM_SLOTS else n - NUM_SLOTS
        avg_ref[n] = jax.lax.dot_general(
            w, values_ref[ksrc], (((1,), (0,)), ((), ())),
            preferred_element_type=jnp.float32) / z
        q3_ref[n] = qn[:, K_DIM * n:K_DIM * (n + 1)]

    avg = avg_ref[...].reshape(N_SLOT_TOT * ROW_BLK, V_DIM)
    q3 = q3_ref[...].reshape(N_SLOT_TOT * ROW_BLK, K_DIM)
    res = jnp.dot(q3, rw_ref[...], preferred_element_type=jnp.float32) + rb_ref[...]
    o = jnp.dot(avg + res, mw_ref[...], preferred_element_type=jnp.float32) + mb_ref[...]
    o = o.reshape(N_SLOT_TOT, ROW_BLK, D_HEAD)
    ok_ = _elu1(o[0:N_HEAD])
    ov = o[N_HEAD:NUM_SLOTS]
    oq = _elu1(o[NUM_SLOTS:])

    c = pl.program_id(0)

    @pl.when(c == 0)
    def _init():
        S_ref[...] = jnp.zeros_like(S_ref)
        z_ref[...] = jnp.zeros_like(z_ref)

    r = jax.lax.broadcasted_iota(jnp.int32, (CHUNK, CHUNK), 0)
    s = jax.lax.broadcasted_iota(jnp.int32, (CHUNK, CHUNK), 1)
    tril = (r >= s).astype(jnp.float32)

    for h in range(N_HEAD):
        kk = ok_[h]
        vv = ov[h]
        qq = oq[h]
        att = jax.lax.dot_general(qq, kk, (((1,), (1,)), ((), ())),
                                  preferred_element_type=jnp.float32)
        att = att * tril
        z0 = z_ref[h, 0:1, :]
        denom = (jnp.sum(att, axis=1, keepdims=True)
                 + jnp.sum(qq * z0, axis=1, keepdims=True))
        out = (jnp.dot(att, vv, preferred_element_type=jnp.float32)
               + jnp.dot(qq, S_ref[h], preferred_element_type=jnp.float32))
        out_ref[:, h, :] = SCALE * out / (denom + EPS)
        S_ref[h] += jax.lax.dot_general(kk, vv, (((0,), (0,)), ((), ())),
                                        preferred_element_type=jnp.float32)
        z_ref[h, 0:1, :] += jnp.sum(kk, axis=0, keepdims=True)


@jax.jit
def _run(x, qproj_w, qproj_b, g576, b576, G, Gt, keys, values,
         res_w, res_b, mem_w, mem_b):
    bs, d_model = x.shape
    n_blk = bs // ROW_BLK

    lo = pl.pallas_call(
        _pkm_body,
        grid=(n_blk,),
        in_specs=[
            pl.BlockSpec((ROW_BLK, d_model), lambda i: (i, 0)),
            pl.BlockSpec((d_model, N_SLOT_TOT * K_DIM), lambda i: (0, 0)),
            pl.BlockSpec((1, N_SLOT_TOT * K_DIM), lambda i: (0, 0)),
            pl.BlockSpec((1, N_SLOT_TOT * K_DIM), lambda i: (0, 0)),
            pl.BlockSpec((1, N_SLOT_TOT * K_DIM), lambda i: (0, 0)),
            pl.BlockSpec((N_SLOT_TOT * K_DIM, N_SLOT_TOT), lambda i: (0, 0)),
            pl.BlockSpec((N_SLOT_TOT, N_SLOT_TOT * K_DIM), lambda i: (0, 0)),
            pl.BlockSpec((NUM_SLOTS, N_KEYS, K_DIM), lambda i: (0, 0, 0)),
            pl.BlockSpec((NUM_SLOTS, N_KEYS, V_DIM), lambda i: (0, 0, 0)),
            pl.BlockSpec((K_DIM, V_DIM), lambda i: (0, 0)),
            pl.BlockSpec((1, V_DIM), lambda i: (0, 0)),
            pl.BlockSpec((V_DIM, D_HEAD), lambda i: (0, 0)),
            pl.BlockSpec((1, D_HEAD), lambda i: (0, 0)),
        ],
        out_specs=pl.BlockSpec((ROW_BLK, N_HEAD, D_HEAD), lambda i: (i, 0, 0)),
        out_shape=jax.ShapeDtypeStruct((bs, N_HEAD, D_HEAD), jnp.float32),
        scratch_shapes=[
            pltpu.VMEM((N_SLOT_TOT, ROW_BLK, V_DIM), jnp.float32),
            pltpu.VMEM((N_SLOT_TOT, ROW_BLK, K_DIM), jnp.float32),
            pltpu.VMEM((NUM_SLOTS, N_KEYS, K_DIM), jnp.float32),
            pltpu.VMEM((ROW_BLK, N_KEYS), jnp.float32),
            pltpu.VMEM((ROW_BLK, N_KEYS), jnp.float32),
            pltpu.VMEM((N_HEAD, D_HEAD, D_HEAD), jnp.float32),
            pltpu.VMEM((N_HEAD, 8, D_HEAD), jnp.float32),
        ],
        compiler_params=pltpu.CompilerParams(
            dimension_semantics=("arbitrary",),
        ),
    )(x, qproj_w, qproj_b, g576, b576, G, Gt, keys, values,
      res_w, res_b, mem_w, mem_b)
    return lo


def kernel(h, qproj_w, qproj_b, ln_g, ln_b, keys, values, res_w, res_b,
           mem_w, mem_b):
    slen, bsz, d_model = h.shape
    x = h.reshape(slen * bsz, d_model)
    cols = N_SLOT_TOT * K_DIM
    grp = jnp.arange(cols, dtype=jnp.int32) // K_DIM
    G = (grp[:, None] == jnp.arange(N_SLOT_TOT, dtype=jnp.int32)[None, :]
         ).astype(jnp.float32)
    Gt = G.T
    g576 = jnp.tile(ln_g, N_SLOT_TOT)[None, :]
    b576 = jnp.tile(ln_b, N_SLOT_TOT)[None, :]
    lo = _run(x, qproj_w, qproj_b[None, :], g576, b576, G, Gt, keys, values,
              res_w, res_b[None, :], mem_w, mem_b[None, :])
    return lo.reshape(bsz, slen, N_HEAD, D_HEAD)


# speedup vs baseline: 28.5675x; 1.0012x over previous
import math

import jax
import jax.numpy as jnp
from jax.experimental import pallas as pl
from jax.experimental.pallas import tpu as pltpu

N_HEAD = 12
K_DIM = 16
V_DIM = 32
D_HEAD = 64
N_KEYS = 512
TOP_K = 8
NUM_SLOTS = 24
N_SLOT_TOT = NUM_SLOTS + N_HEAD
SCALE = 1.0 / math.sqrt(D_HEAD)
EPS = 1e-5

ROW_BLK = 256
CHUNK = 256


def _elu1(x):
    return jnp.where(x > 0, x + 1.0, jnp.exp(x))


def _pkm_body(x_ref, w_ref, b_ref, g_ref, bln_ref, G_ref, Gt_ref,
              keys_ref, values_ref, rw_ref, rb_ref, mw_ref, mb_ref, out_ref,
              avg_ref, q3_ref, kn_ref, scA_ref, scB_ref, S_ref, z_ref):
    x = x_ref[...]
    q = jnp.dot(x, w_ref[...], preferred_element_type=jnp.float32) + b_ref[...]
    G = G_ref[...]
    s1 = jnp.dot(q, G, preferred_element_type=jnp.float32) * (1.0 / K_DIM)
    s2 = jnp.dot(q * q, G, preferred_element_type=jnp.float32) * (1.0 / K_DIM)
    var = s2 - s1 * s1
    Gt = Gt_ref[...]
    mu_b = jnp.dot(s1, Gt, preferred_element_type=jnp.float32)
    var_b = jnp.dot(var, Gt, preferred_element_type=jnp.float32)
    qn = (q - mu_b) * jax.lax.rsqrt(var_b + 1e-5) * g_ref[...] + bln_ref[...]

    for k in range(NUM_SLOTS):
        kk = keys_ref[k]
        kn_ref[k] = kk * jax.lax.rsqrt(jnp.sum(kk * kk, axis=1, keepdims=True))

    Q = N_KEYS // 4
    sc_bufs = (scA_ref, scB_ref)

    def _scores(i):
        ksrc = i if i < NUM_SLOTS else i - NUM_SLOTS
        qn_i = qn[:, K_DIM * i:K_DIM * (i + 1)]
        sc_bufs[i % 2][...] = jax.lax.dot_general(
            qn_i, kn_ref[ksrc], (((1,), (1,)), ((), ())),
            preferred_element_type=jnp.float32)

    _scores(0)
    for n in range(N_SLOT_TOT):
        if n + 1 < N_SLOT_TOT:
            _scores(n + 1)
        sc = sc_bufs[n % 2][...]
        a, b = sc[:, 0:Q], sc[:, Q:2 * Q]
        cc, d = sc[:, 2 * Q:3 * Q], sc[:, 3 * Q:]
        ab_h, ab_l = jnp.maximum(a, b), jnp.minimum(a, b)
        cd_h, cd_l = jnp.maximum(cc, d), jnp.minimum(cc, d)
        t1 = jnp.maximum(ab_h, cd_h)
        m_h = jnp.minimum(ab_h, cd_h)
        t4 = jnp.minimum(ab_l, cd_l)
        m_l = jnp.maximum(ab_l, cd_l)
        t2 = jnp.maximum(m_h, m_l)
        t3 = jnp.minimum(m_h, m_l)
        m0 = jnp.max(t1, axis=1, keepdims=True)
        thr = m0
        for it in range(TOP_K - 1):
            e = t1 >= thr
            t1 = jnp.where(e, t2, t1)
            if it < TOP_K - 2:
                t2 = jnp.where(e, t3, t2)
            if it < TOP_K - 3:
                t3 = jnp.where(e, t4, t3)
            if it < TOP_K - 4:
                t4 = jnp.where(e, -1e30, t4)
            thr = jnp.max(t1, axis=1, keepdims=True)
        w = jnp.where(sc >= thr, jnp.exp(sc - m0), 0.0)
        z = jnp.sum(w, axis=1, keepdims=True)
        ksrc = n if n < NUM_SLOTS else n - NUM_SLOTS
        avg_ref[n] = jax.lax.dot_general(
            w, values_ref[ksrc], (((1,), (0,)), ((), ())),
            preferred_element_type=jnp.float32) / z
        q3_ref[n] = qn[:, K_DIM * n:K_DIM * (n + 1)]

    avg = avg_ref[...].reshape(N_SLOT_TOT * ROW_BLK, V_DIM)
    q3 = q3_ref[...].reshape(N_SLOT_TOT * ROW_BLK, K_DIM)
    res = jnp.dot(q3, rw_ref[...], preferred_element_type=jnp.float32) + rb_ref[...]
    o = jnp.dot(avg + res, mw_ref[...], preferred_element_type=jnp.float32) + mb_ref[...]
    o = o.reshape(N_SLOT_TOT, ROW_BLK, D_HEAD)
    ok_ = _elu1(o[0:N_HEAD])
    ov = o[N_HEAD:NUM_SLOTS]
    oq = _elu1(o[NUM_SLOTS:])

    c = pl.program_id(0)

    @pl.when(c == 0)
    def _init():
        S_ref[...] = jnp.zeros_like(S_ref)
        z_ref[...] = jnp.zeros_like(z_ref)

    r = jax.lax.broadcasted_iota(jnp.int32, (CHUNK, CHUNK), 0)
    s = jax.lax.broadcasted_iota(jnp.int32, (CHUNK, CHUNK), 1)
    tril = (r >= s).astype(jnp.float32)

    for h in range(N_HEAD):
        kk = ok_[h]
        vv = ov[h]
        qq = oq[h]
        att = jax.lax.dot_general(qq, kk, (((1,), (1,)), ((), ())),
                                  preferred_element_type=jnp.float32)
        att = att * tril
        z0 = z_ref[h, 0:1, :]
        denom = (jnp.sum(att, axis=1, keepdims=True)
                 + jnp.sum(qq * z0, axis=1, keepdims=True))
        out = (jnp.dot(att, vv, preferred_element_type=jnp.float32)
               + jnp.dot(qq, S_ref[h], preferred_element_type=jnp.float32))
        out_ref[:, h, :] = SCALE * out / (denom + EPS)
        S_ref[h] += jax.lax.dot_general(kk, vv, (((0,), (0,)), ((), ())),
                                        preferred_element_type=jnp.float32)
        z_ref[h, 0:1, :] += jnp.sum(kk, axis=0, keepdims=True)


@jax.jit
def _run(x, qproj_w, qproj_b, g576, b576, G, Gt, keys, values,
         res_w, res_b, mem_w, mem_b):
    bs, d_model = x.shape
    n_blk = bs // ROW_BLK

    lo = pl.pallas_call(
        _pkm_body,
        grid=(n_blk,),
        in_specs=[
            pl.BlockSpec((ROW_BLK, d_model), lambda i: (i, 0)),
            pl.BlockSpec((d_model, N_SLOT_TOT * K_DIM), lambda i: (0, 0)),
            pl.BlockSpec((1, N_SLOT_TOT * K_DIM), lambda i: (0, 0)),
            pl.BlockSpec((1, N_SLOT_TOT * K_DIM), lambda i: (0, 0)),
            pl.BlockSpec((1, N_SLOT_TOT * K_DIM), lambda i: (0, 0)),
            pl.BlockSpec((N_SLOT_TOT * K_DIM, N_SLOT_TOT), lambda i: (0, 0)),
            pl.BlockSpec((N_SLOT_TOT, N_SLOT_TOT * K_DIM), lambda i: (0, 0)),
            pl.BlockSpec((NUM_SLOTS, N_KEYS, K_DIM), lambda i: (0, 0, 0)),
            pl.BlockSpec((NUM_SLOTS, N_KEYS, V_DIM), lambda i: (0, 0, 0)),
            pl.BlockSpec((K_DIM, V_DIM), lambda i: (0, 0)),
            pl.BlockSpec((1, V_DIM), lambda i: (0, 0)),
            pl.BlockSpec((V_DIM, D_HEAD), lambda i: (0, 0)),
            pl.BlockSpec((1, D_HEAD), lambda i: (0, 0)),
        ],
        out_specs=pl.BlockSpec((ROW_BLK, N_HEAD, D_HEAD), lambda i: (i, 0, 0)),
        out_shape=jax.ShapeDtypeStruct((bs, N_HEAD, D_HEAD), jnp.float32),
        scratch_shapes=[
            pltpu.VMEM((N_SLOT_TOT, ROW_BLK, V_DIM), jnp.float32),
            pltpu.VMEM((N_SLOT_TOT, ROW_BLK, K_DIM), jnp.float32),
            pltpu.VMEM((NUM_SLOTS, N_KEYS, K_DIM), jnp.float32),
            pltpu.VMEM((ROW_BLK, N_KEYS), jnp.float32),
            pltpu.VMEM((ROW_BLK, N_KEYS), jnp.float32),
            pltpu.VMEM((N_HEAD, D_HEAD, D_HEAD), jnp.float32),
            pltpu.VMEM((N_HEAD, 8, D_HEAD), jnp.float32),
        ],
        compiler_params=pltpu.CompilerParams(
            dimension_semantics=("arbitrary",),
        ),
    )(x, qproj_w, qproj_b, g576, b576, G, Gt, keys, values,
      res_w, res_b, mem_w, mem_b)
    return lo


def kernel(h, qproj_w, qproj_b, ln_g, ln_b, keys, values, res_w, res_b,
           mem_w, mem_b):
    slen, bsz, d_model = h.shape
    x = h.reshape(slen * bsz, d_model)
    cols = N_SLOT_TOT * K_DIM
    grp = jnp.arange(cols, dtype=jnp.int32) // K_DIM
    G = (grp[:, None] == jnp.arange(N_SLOT_TOT, dtype=jnp.int32)[None, :]
         ).astype(jnp.float32)
    Gt = G.T
    g576 = jnp.tile(ln_g, N_SLOT_TOT)[None, :]
    b576 = jnp.tile(ln_b, N_SLOT_TOT)[None, :]
    lo = _run(x, qproj_w, qproj_b[None, :], g576, b576, G, Gt, keys, values,
              res_w, res_b[None, :], mem_w, mem_b[None, :])
    return lo.reshape(bsz, slen, N_HEAD, D_HEAD)
